# baseline (speedup 1.0000x reference)
import jax
import jax.numpy as jnp
from jax import lax
from jax.experimental import pallas as pl
from jax.experimental.pallas import tpu as pltpu
from jax.experimental.pallas import tpu_sc as plsc

CTX_DIM = 512
CONTEXT_LEN = 77
N_CTX = 16
N_TOK = CONTEXT_LEN - N_CTX
N_WORK = 4


def _sc_body(tok_hbm, table_hbm, ctx_hbm, out_hbm, tokv, gv, cv, sem):
    wid = lax.axis_index("s")

    @pl.when(wid < N_WORK)
    def _():
        pltpu.sync_copy(tok_hbm, tokv)
        slot = lax.iota(jnp.int32, 16) + 16 * wid
        row = jnp.where(slot == 0, 0,
                        jnp.minimum(slot + N_CTX, CONTEXT_LEN - 1))
        col = jnp.minimum(slot, N_TOK - 1)
        src = plsc.load_gather(tokv, [jnp.zeros((16,), jnp.int32), col])
        pltpu.async_copy(table_hbm.at[src], gv, sem).wait()
        pltpu.async_copy(gv, out_hbm.at[row], sem).wait()

    @pl.when(wid == N_WORK)
    def _():
        pltpu.sync_copy(ctx_hbm, cv)
        cidx = lax.iota(jnp.int32, 16) + 1
        pltpu.async_copy(cv, out_hbm.at[cidx], sem).wait()


@jax.jit
def _sc_call(tok, token_embedding, ctx):
    mesh = plsc.VectorSubcoreMesh(
        core_axis_name="c", subcore_axis_name="s", num_cores=1)
    return pl.kernel(
        _sc_body,
        out_type=jax.ShapeDtypeStruct((CONTEXT_LEN, CTX_DIM), jnp.float32),
        mesh=mesh,
        scratch_types=[
            pltpu.VMEM((1, CONTEXT_LEN), jnp.int32),
            pltpu.VMEM((16, CTX_DIM), jnp.float32),
            pltpu.VMEM((N_CTX, CTX_DIM), jnp.float32),
            pltpu.SemaphoreType.DMA,
        ],
        compiler_params=pltpu.CompilerParams(needs_layout_passes=False),
    )(tok, token_embedding, ctx)


def kernel(tokenized_prompts, token_embedding, ctx):
    return _sc_call(tokenized_prompts.astype(jnp.int32), token_embedding, ctx)
